# baseline (speedup 1.0000x reference)
import functools

import jax
import jax.numpy as jnp
from jax import lax
from jax.experimental import pallas as pl
from jax.experimental.pallas import tpu as pltpu
from jax.experimental.pallas import tpu_sc as plsc

N_VOCAB = 49408
N_EMBD = 768
N_TOKEN = 77
BATCH = 1024

NC = 2
NS = 16
NW = NC * NS
LANES = 16

FLAT = BATCH * N_TOKEN
ROWS_W = FLAT // NW
CHUNK = 16
NCH = ROWS_W // CHUNK
DGROUPS = N_EMBD // LANES

_mesh = plsc.VectorSubcoreMesh(
    core_axis_name="c", subcore_axis_name="s", num_cores=NC, num_subcores=NS
)


@functools.partial(
    pl.kernel,
    out_type=jax.ShapeDtypeStruct((FLAT, N_EMBD), jnp.float32),
    mesh=_mesh,
    scratch_types=[
        pltpu.VMEM((ROWS_W,), jnp.int32),
        pltpu.VMEM((N_TOKEN * N_EMBD,), jnp.float32),
        pltpu.VMEM((2, CHUNK, N_EMBD), jnp.float32),
        pltpu.SemaphoreType.DMA,
        pltpu.SemaphoreType.DMA,
        pltpu.SemaphoreType.DMA,
        pltpu.SemaphoreType.DMA,
    ],
)
def _emb_lookup(tok_hbm, tab_hbm, pos_hbm, out_hbm,
                idx_v, pos_v, rows_v, g0, g1, s0, s1):
    gsem = (g0, g1)
    ssem = (s0, s1)
    wid = lax.axis_index("s") * NC + lax.axis_index("c")
    base = wid * ROWS_W

    pltpu.sync_copy(tok_hbm.at[pl.ds(base, ROWS_W)], idx_v)
    pltpu.sync_copy(pos_hbm, pos_v)

    def gather_desc(c, b):
        return pltpu.make_async_copy(
            tab_hbm.at[idx_v.at[pl.ds(c * CHUNK, CHUNK)]], rows_v.at[b], gsem[b]
        )

    def scatter_desc(c, b):
        return pltpu.make_async_copy(
            rows_v.at[b], out_hbm.at[pl.ds(base + c * CHUNK, CHUNK)], ssem[b]
        )

    gather_desc(0, 0).start()

    def pair_body(jj, carry):
        for b in range(2):
            j = jj * 2 + b
            bn = 1 - b

            @pl.when(j >= 1)
            def _():
                scatter_desc(j - 1, bn).wait()

            @pl.when(j + 1 < NCH)
            def _():
                gather_desc(j + 1, bn).start()

            gather_desc(j, b).wait()
            row0 = base + j * CHUNK
            for i in range(CHUNK):
                p = lax.rem(row0 + i, N_TOKEN)
                pbase = p * N_EMBD
                for d in range(DGROUPS):
                    x = pos_v[pl.ds(pbase + d * LANES, LANES)]
                    plsc.addupdate(rows_v.at[b, i, pl.ds(d * LANES, LANES)], x)

            scatter_desc(j, b).start()
        return carry

    lax.fori_loop(0, NCH // 2, pair_body, 0)

    scatter_desc(NCH - 1, (NCH - 1) % 2).wait()


def kernel(tokens, token_embedding, position_value):
    tok = tokens.reshape(-1).astype(jnp.int32)
    pos = position_value.reshape(-1)
    out = _emb_lookup(tok, token_embedding, pos)
    return out.reshape(BATCH, N_TOKEN, N_EMBD)


# speedup vs baseline: 1.3744x; 1.3744x over previous
import functools

import jax
import jax.numpy as jnp
from jax import lax
from jax.experimental import pallas as pl
from jax.experimental.pallas import tpu as pltpu
from jax.experimental.pallas import tpu_sc as plsc

N_VOCAB = 49408
N_EMBD = 768
N_TOKEN = 77
BATCH = 1024

NC = 2
NS = 16
NW = NC * NS
LANES = 16

FLAT = BATCH * N_TOKEN
ROWS_W = FLAT // NW
CHUNK = 16
NCH = ROWS_W // CHUNK
DGROUPS = N_EMBD // LANES

_mesh = plsc.VectorSubcoreMesh(
    core_axis_name="c", subcore_axis_name="s", num_cores=NC, num_subcores=NS
)


@functools.partial(
    pl.kernel,
    out_type=jax.ShapeDtypeStruct((FLAT, N_EMBD), jnp.float32),
    mesh=_mesh,
    scratch_types=[
        pltpu.VMEM((ROWS_W,), jnp.int32),
        pltpu.VMEM((N_TOKEN * N_EMBD,), jnp.float32),
        pltpu.VMEM((2, CHUNK, N_EMBD), jnp.float32),
        pltpu.SemaphoreType.DMA,
        pltpu.SemaphoreType.DMA,
        pltpu.SemaphoreType.DMA,
        pltpu.SemaphoreType.DMA,
    ],
)
def _emb_lookup(tok_hbm, tab_hbm, pos_hbm, out_hbm,
                idx_v, pos_v, rows_v, g0, g1, s0, s1):
    gsem = (g0, g1)
    ssem = (s0, s1)
    wid = lax.axis_index("s") * NC + lax.axis_index("c")
    base = wid * ROWS_W

    pltpu.sync_copy(tok_hbm.at[pl.ds(base, ROWS_W)], idx_v)
    pltpu.sync_copy(pos_hbm, pos_v)

    def gather_desc(c, b):
        return pltpu.make_async_copy(
            tab_hbm.at[idx_v.at[pl.ds(c * CHUNK, CHUNK)]], rows_v.at[b], gsem[b]
        )

    def scatter_desc(c, b):
        return pltpu.make_async_copy(
            rows_v.at[b], out_hbm.at[pl.ds(base + c * CHUNK, CHUNK)], ssem[b]
        )

    gather_desc(0, 0).start()

    def pair_body(jj, carry):
        for b in range(2):
            j = jj * 2 + b
            bn = 1 - b

            @pl.when(j >= 1)
            def _():
                scatter_desc(j - 1, bn).wait()

            @pl.when(j + 1 < NCH)
            def _():
                gather_desc(j + 1, bn).start()

            gather_desc(j, b).wait()
            row0 = base + j * CHUNK
            for i in range(CHUNK):
                p = lax.rem(row0 + i, N_TOKEN)
                pbase = p * N_EMBD
                ngrp = DGROUPS // 8
                xs = [pos_v[pl.ds(pbase + d * LANES, LANES)] for d in range(8)]
                for g in range(1, ngrp + 1):
                    if g < ngrp:
                        ys = [pos_v[pl.ds(pbase + (8 * g + d) * LANES, LANES)]
                              for d in range(8)]
                    for d in range(8):
                        plsc.addupdate(
                            rows_v.at[b, i,
                                      pl.ds((8 * (g - 1) + d) * LANES, LANES)],
                            xs[d])
                    if g < ngrp:
                        xs = ys

            scatter_desc(j, b).start()
        return carry

    lax.fori_loop(0, NCH // 2, pair_body, 0)

    scatter_desc(NCH - 1, (NCH - 1) % 2).wait()


def kernel(tokens, token_embedding, position_value):
    tok = tokens.reshape(-1).astype(jnp.int32)
    pos = position_value.reshape(-1)
    out = _emb_lookup(tok, token_embedding, pos)
    return out.reshape(BATCH, N_TOKEN, N_EMBD)


# speedup vs baseline: 1.5365x; 1.1179x over previous
import functools

import jax
import jax.numpy as jnp
from jax import lax
from jax.experimental import pallas as pl
from jax.experimental.pallas import tpu as pltpu
from jax.experimental.pallas import tpu_sc as plsc

N_VOCAB = 49408
N_EMBD = 768
N_TOKEN = 77
BATCH = 1024

NC = 2
NS = 16
NW = NC * NS
LANES = 16

T_PAD = 80
H0 = 40
H1 = N_TOKEN - H0
B_W = BATCH // NW
DGROUPS = N_EMBD // LANES

_mesh = plsc.VectorSubcoreMesh(
    core_axis_name="c", subcore_axis_name="s", num_cores=NC, num_subcores=NS
)


@functools.partial(
    pl.kernel,
    out_type=jax.ShapeDtypeStruct((BATCH, N_TOKEN, N_EMBD), jnp.float32),
    mesh=_mesh,
    compiler_params=pltpu.CompilerParams(use_tc_tiling_on_sc=False),
    scratch_types=[
        pltpu.VMEM((B_W * T_PAD,), jnp.int32),
        pltpu.VMEM((N_TOKEN * N_EMBD,), jnp.float32),
        pltpu.VMEM((N_TOKEN, N_EMBD), jnp.float32),
        pltpu.SemaphoreType.DMA,
        pltpu.SemaphoreType.DMA,
        pltpu.SemaphoreType.DMA,
        pltpu.SemaphoreType.DMA,
    ],
)
def _emb_lookup(tok_hbm, tab_hbm, pos_hbm, out_hbm,
                idx_v, pos_v, rows_v, g0, g1, s0, s1):
    gsem = (g0, g1)
    ssem = (s0, s1)
    wid = lax.axis_index("s") * NC + lax.axis_index("c")
    b0 = wid * B_W

    pltpu.sync_copy(tok_hbm.at[pl.ds(b0 * T_PAD, B_W * T_PAD)], idx_v)
    pltpu.sync_copy(pos_hbm, pos_v)

    def gather_desc(b, h):
        t0, cnt = (0, H0) if h == 0 else (H0, H1)
        return pltpu.make_async_copy(
            tab_hbm.at[idx_v.at[pl.ds(b * T_PAD + t0, cnt)]],
            rows_v.at[pl.ds(t0, cnt)],
            gsem[h],
        )

    def scatter_desc(b, h):
        t0, cnt = (0, H0) if h == 0 else (H0, H1)
        return pltpu.make_async_copy(
            rows_v.at[pl.ds(t0, cnt)],
            out_hbm.at[b0 + b, pl.ds(t0, cnt)],
            ssem[h],
        )

    def add_positions(h):
        t0, cnt = (0, H0) if h == 0 else (H0, H1)

        def row_body(t, carry):
            pbase = t * N_EMBD
            ngrp = DGROUPS // 8
            xs = [pos_v[pl.ds(pbase + d * LANES, LANES)] for d in range(8)]
            for g in range(1, ngrp + 1):
                if g < ngrp:
                    ys = [pos_v[pl.ds(pbase + (8 * g + d) * LANES, LANES)]
                          for d in range(8)]
                for d in range(8):
                    plsc.addupdate(
                        rows_v.at[t, pl.ds((8 * (g - 1) + d) * LANES, LANES)],
                        xs[d])
                if g < ngrp:
                    xs = ys
            return carry

        lax.fori_loop(t0, t0 + cnt, row_body, 0)

    gather_desc(0, 0).start()

    def batch_body(b, carry):
        @pl.when(b >= 1)
        def _():
            scatter_desc(b - 1, 1).wait()

        gather_desc(b, 1).start()
        gather_desc(b, 0).wait()
        add_positions(0)
        scatter_desc(b, 0).start()

        scatter_desc(b, 0).wait()

        @pl.when(b < B_W - 1)
        def _():
            gather_desc(b + 1, 0).start()

        gather_desc(b, 1).wait()
        add_positions(1)
        scatter_desc(b, 1).start()
        return carry

    lax.fori_loop(0, B_W, batch_body, 0)

    scatter_desc(B_W - 1, 1).wait()


def kernel(tokens, token_embedding, position_value):
    tok = jnp.pad(tokens.astype(jnp.int32), ((0, 0), (0, T_PAD - N_TOKEN)))
    pos = position_value.reshape(-1)
    return _emb_lookup(tok.reshape(-1), token_embedding, pos)
